# baseline (speedup 1.0000x reference)
import dataclasses
import functools

import jax
import jax.numpy as jnp
from jax import lax
from jax.experimental import pallas as pl
from jax.experimental.pallas import tpu as pltpu
from jax.experimental.pallas import tpu_sc as plsc

_N = 256
_MARGIN = 1.0
_L = 16
_NW = 32


def _dist_kernel(e_ref, d_ref):
    e = e_ref[...]
    g = jax.lax.dot_general(e, e, (((1,), (1,)), ((), ())),
                            preferred_element_type=jnp.float32)
    sq = jnp.sum(e * e, axis=1, keepdims=True)
    d_ref[...] = sq + jnp.transpose(sq) - 2.0 * g


def _sc_body(dist_hbm, tgt_hbm, part_hbm, drow_v, tgt_v, plist_v,
             obuf_v, sem):
    cid = lax.axis_index("c")
    sid = lax.axis_index("s")
    w = sid * 2 + cid
    w4 = w * 4

    c1 = pltpu.async_copy(tgt_hbm, tgt_v, sem)
    c2 = pltpu.async_copy(dist_hbm.at[pl.ds(w4, 4)],
                          drow_v.at[pl.ds(0, 4)], sem)
    c3 = pltpu.async_copy(dist_hbm.at[pl.ds(252 - w4, 4)],
                          drow_v.at[pl.ds(4, 4)], sem)
    c1.wait()

    lanes = lax.iota(jnp.int32, _L)
    big = jnp.float32(1e30)
    accv = jnp.zeros((_L,), jnp.float32)
    cntv = jnp.zeros((_L,), jnp.int32)
    tchunks = [tgt_v[pl.ds(c * _L, _L)] for c in range(_N // _L)]
    anchors = []
    for j in range(8):
        if j < 4:
            anchors.append(w4 + j)
        else:
            anchors.append(252 - w4 + (j - 4))
    ta_vecs = [plsc.load_gather(tgt_v, [jnp.full((_L,), a, jnp.int32)])
               for a in anchors]

    npos_list = []
    for j in range(8):
        a = anchors[j]
        ta_vec = ta_vecs[j]
        npos_vec = jnp.zeros((_L,), jnp.int32)
        nneg_vec = jnp.zeros((_L,), jnp.int32)
        for c in range(_N // _L):
            tgtc = tchunks[c]
            negm = tgtc != ta_vec
            nneg_vec = nneg_vec + plsc.all_reduce_population_count(negm)
            if j >= 4 and c < 8:
                continue
            pidx = lanes + (c * _L)
            posm = (tgtc == ta_vec) & (pidx > a)
            cpos = plsc.cumsum(posm.astype(jnp.int32))
            plsc.store_scatter(plist_v, [j * _N + npos_vec + cpos - 1],
                               pidx, mask=posm)
            npos_vec = npos_vec + plsc.all_reduce_population_count(posm)
        cntv = cntv + npos_vec * nneg_vec
        npos_list.append(jnp.max(npos_vec))

    c2.wait()
    c3.wait()
    for j in range(8):
        ta_vec = ta_vecs[j]
        dms = []
        for c in range(_N // _L):
            drc = drow_v[j, pl.ds(c * _L, _L)]
            dms.append(jnp.where(tchunks[c] != ta_vec, drc, big))

        def pair_body(i, acc, j=j, dms=dms):
            pvec = plsc.load_gather(plist_v,
                                    [jnp.full((_L,), j * _N, jnp.int32) + i])
            t = plsc.load_gather(drow_v, [jnp.full((_L,), j, jnp.int32), pvec])
            t = t + jnp.float32(_MARGIN)
            for c in range(_N // _L):
                acc = acc + jnp.maximum(t - dms[c], 0.0)
            return acc

        accv = lax.fori_loop(0, npos_list[j], pair_body, accv)

    obuf_v[pl.ds(0, _L)] = accv
    obuf_v[pl.ds(_L, _L)] = jnp.where(lanes == 0, cntv, 0).astype(jnp.float32)
    pltpu.sync_copy(obuf_v, part_hbm.at[w])


def _fin_kernel(part_ref, mean_ref, cnt_ref):
    p = part_ref[...]
    total = jnp.sum(p[:, :_L])
    cnt = jnp.sum(p[:, _L:])
    mean_ref[0, 0] = total / cnt
    cnt_ref[0, 0] = cnt.astype(jnp.int32)


@jax.jit
def kernel(embeddings, target, max_score):
    dist = pl.pallas_call(
        _dist_kernel,
        out_shape=jax.ShapeDtypeStruct((_N, _N), jnp.float32),
    )(embeddings)

    mesh = plsc.VectorSubcoreMesh(core_axis_name="c", subcore_axis_name="s")
    cp = pltpu.CompilerParams()
    if "needs_layout_passes" in pltpu.CompilerParams.__dataclass_fields__:
        cp = dataclasses.replace(cp, needs_layout_passes=False)
    sc_loss = functools.partial(
        pl.kernel,
        mesh=mesh,
        compiler_params=cp,
        out_type=jax.ShapeDtypeStruct((_NW, 2 * _L), jnp.float32),
        scratch_types=[
            pltpu.VMEM((8, _N), jnp.float32),
            pltpu.VMEM((_N,), jnp.int32),
            pltpu.VMEM((8 * _N,), jnp.int32),
            pltpu.VMEM((2 * _L,), jnp.float32),
            pltpu.SemaphoreType.DMA,
        ],
    )(_sc_body)
    part = sc_loss(dist, target)

    mean, cnt = pl.pallas_call(
        _fin_kernel,
        out_shape=[
            jax.ShapeDtypeStruct((1, 1), jnp.float32),
            jax.ShapeDtypeStruct((1, 1), jnp.int32),
        ],
        out_specs=[
            pl.BlockSpec(memory_space=pltpu.SMEM),
            pl.BlockSpec(memory_space=pltpu.SMEM),
        ],
    )(part)

    return mean[0, 0], cnt[0, 0]


# speedup vs baseline: 1.0194x; 1.0194x over previous
import dataclasses
import functools

import jax
import jax.numpy as jnp
from jax import lax
from jax.experimental import pallas as pl
from jax.experimental.pallas import tpu as pltpu
from jax.experimental.pallas import tpu_sc as plsc

_N = 256
_MARGIN = 1.0
_L = 16
_NW = 32


def _dist_kernel(e_ref, d_ref):
    e = e_ref[...]
    g = jax.lax.dot_general(e, e, (((1,), (1,)), ((), ())),
                            preferred_element_type=jnp.float32)
    sq = jnp.sum(e * e, axis=1, keepdims=True)
    d_ref[...] = sq + jnp.transpose(sq) - 2.0 * g


def _sc_body(dist_hbm, tgt_hbm, part_hbm, drow_v, tgt_v, plist_v,
             obuf_v, sem):
    cid = lax.axis_index("c")
    sid = lax.axis_index("s")
    w = sid * 2 + cid
    w4 = w * 4

    c1 = pltpu.async_copy(tgt_hbm, tgt_v, sem)
    c2 = pltpu.async_copy(dist_hbm.at[pl.ds(w4, 4)],
                          drow_v.at[pl.ds(0, 4)], sem)
    c3 = pltpu.async_copy(dist_hbm.at[pl.ds(252 - w4, 4)],
                          drow_v.at[pl.ds(4, 4)], sem)
    c1.wait()
    c2.wait()
    c3.wait()

    lanes = lax.iota(jnp.int32, _L)
    big = jnp.float32(1e30)
    accv = jnp.zeros((_L,), jnp.float32)
    cntv = jnp.zeros((_L,), jnp.int32)
    tchunks = [tgt_v[pl.ds(c * _L, _L)] for c in range(_N // _L)]

    for j in range(8):
        if j < 4:
            a = w4 + j
        else:
            a = 252 - w4 + (j - 4)
        ta_vec = plsc.load_gather(tgt_v, [jnp.full((_L,), a, jnp.int32)])

        npos_vec = jnp.zeros((_L,), jnp.int32)
        nneg_vec = jnp.zeros((_L,), jnp.int32)
        dms = []
        for c in range(_N // _L):
            tgtc = tchunks[c]
            drc = drow_v[j, pl.ds(c * _L, _L)]
            negm = tgtc != ta_vec
            dms.append(jnp.where(negm, drc, big))
            nneg_vec = nneg_vec + plsc.all_reduce_population_count(negm)
            if j >= 4 and c < 8:
                continue
            pidx = lanes + (c * _L)
            posm = (tgtc == ta_vec) & (pidx > a)
            cpos = plsc.cumsum(posm.astype(jnp.int32))
            plsc.store_scatter(plist_v, [npos_vec + cpos - 1], pidx,
                               mask=posm)
            npos_vec = npos_vec + plsc.all_reduce_population_count(posm)

        cntv = cntv + npos_vec * nneg_vec
        npos_s = jnp.max(npos_vec)

        def pair_body(i, acc, j=j, dms=dms):
            pvec = plsc.load_gather(plist_v, [jnp.full((_L,), i, jnp.int32)])
            t = plsc.load_gather(drow_v, [jnp.full((_L,), j, jnp.int32), pvec])
            t = t + jnp.float32(_MARGIN)
            for c in range(_N // _L):
                acc = acc + jnp.maximum(t - dms[c], 0.0)
            return acc

        accv = lax.fori_loop(0, npos_s, pair_body, accv)

    obuf_v[pl.ds(0, _L)] = accv
    obuf_v[pl.ds(_L, _L)] = jnp.where(lanes == 0, cntv, 0).astype(jnp.float32)
    pltpu.sync_copy(obuf_v, part_hbm.at[w])


def _fin_kernel(part_ref, mean_ref, cnt_ref):
    p = part_ref[...]
    total = jnp.sum(p[:, :_L])
    cnt = jnp.sum(p[:, _L:])
    mean_ref[0, 0] = total / cnt
    cnt_ref[0, 0] = cnt.astype(jnp.int32)


@jax.jit
def kernel(embeddings, target, max_score):
    dist = pl.pallas_call(
        _dist_kernel,
        out_shape=jax.ShapeDtypeStruct((_N, _N), jnp.float32),
    )(embeddings)

    mesh = plsc.VectorSubcoreMesh(core_axis_name="c", subcore_axis_name="s")
    cp = pltpu.CompilerParams()
    if "needs_layout_passes" in pltpu.CompilerParams.__dataclass_fields__:
        cp = dataclasses.replace(cp, needs_layout_passes=False)
    sc_loss = functools.partial(
        pl.kernel,
        mesh=mesh,
        compiler_params=cp,
        out_type=jax.ShapeDtypeStruct((_NW, 2 * _L), jnp.float32),
        scratch_types=[
            pltpu.VMEM((8, _N), jnp.float32),
            pltpu.VMEM((_N,), jnp.int32),
            pltpu.VMEM((8 * _N,), jnp.int32),
            pltpu.VMEM((2 * _L,), jnp.float32),
            pltpu.SemaphoreType.DMA,
        ],
    )(_sc_body)
    part = sc_loss(dist, target)

    mean, cnt = pl.pallas_call(
        _fin_kernel,
        out_shape=[
            jax.ShapeDtypeStruct((1, 1), jnp.float32),
            jax.ShapeDtypeStruct((1, 1), jnp.int32),
        ],
        out_specs=[
            pl.BlockSpec(memory_space=pltpu.SMEM),
            pl.BlockSpec(memory_space=pltpu.SMEM),
        ],
    )(part)

    return mean[0, 0], cnt[0, 0]
